# baseline (speedup 1.0000x reference)
import functools

import jax
import jax.numpy as jnp
from jax import lax
from jax.experimental import pallas as pl
from jax.experimental.pallas import tpu as pltpu
from jax.experimental.pallas import tpu_sc as plsc

_N = 1048576 * 3 * 3
_NC = 2
_NS = 16
_NW = _NC * _NS
_PER_W = _N // _NW
_CHUNK = 24576
_NCHUNK = _PER_W // _CHUNK
_LANES = 16
_UNROLL = 8
_STEP = _LANES * _UNROLL


def _tec_body(x_hbm, out_hbm, in0, in1, out0, out1,
              isem0, isem1, osem0, osem1):
    wid = lax.axis_index("s") * _NC + lax.axis_index("c")
    base = wid * _PER_W
    inbuf = (in0, in1)
    outbuf = (out0, out1)
    isem = (isem0, isem1)
    osem = (osem0, osem1)

    def load(c):
        b = c % 2
        return pltpu.async_copy(
            x_hbm.at[pl.ds(base + c * _CHUNK, _CHUNK)], inbuf[b], isem[b])

    def store(c):
        b = c % 2
        return pltpu.async_copy(
            outbuf[b], out_hbm.at[pl.ds(base + c * _CHUNK, _CHUNK)], osem[b])

    loads = {0: load(0)}
    if _NCHUNK > 1:
        loads[1] = load(1)
    stores = {}
    for c in range(_NCHUNK):
        b = c % 2
        loads[c].wait()
        if c >= 2:
            stores[c - 2].wait()
        src = inbuf[b]
        dst = outbuf[b]

        def step(i, carry):
            off = i * _STEP
            for u in range(_UNROLL):
                o = off + u * _LANES
                v = src[pl.ds(o, _LANES)]
                dst[pl.ds(o, _LANES)] = jnp.where(
                    v != 0.0, jnp.float32(2.0), v)
            return carry

        lax.fori_loop(0, _CHUNK // _STEP, step, jnp.int32(0))
        stores[c] = store(c)
        if c + 2 < _NCHUNK:
            loads[c + 2] = load(c + 2)
    for c in range(max(0, _NCHUNK - 2), _NCHUNK):
        stores[c].wait()


@functools.partial(
    pl.kernel,
    mesh=plsc.VectorSubcoreMesh(core_axis_name="c", subcore_axis_name="s"),
    out_type=jax.ShapeDtypeStruct((_N,), jnp.float32),
    scratch_types=[
        pltpu.VMEM((_CHUNK,), jnp.float32),
        pltpu.VMEM((_CHUNK,), jnp.float32),
        pltpu.VMEM((_CHUNK,), jnp.float32),
        pltpu.VMEM((_CHUNK,), jnp.float32),
        pltpu.SemaphoreType.DMA,
        pltpu.SemaphoreType.DMA,
        pltpu.SemaphoreType.DMA,
        pltpu.SemaphoreType.DMA,
    ],
)
def _sc_mask_set(x_hbm, out_hbm, *scratch):
    _tec_body(x_hbm, out_hbm, *scratch)


def kernel(x):
    flat = x.reshape(_N)
    out = _sc_mask_set(flat)
    return out.reshape(x.shape)


# speedup vs baseline: 1.0032x; 1.0032x over previous
import functools

import jax
import jax.numpy as jnp
from jax import lax
from jax.experimental import pallas as pl
from jax.experimental.pallas import tpu as pltpu
from jax.experimental.pallas import tpu_sc as plsc

_N = 1048576 * 3 * 3
_COLS = 128
_ROWS = _N // _COLS
_NC = 2
_NS = 16
_NW = _NC * _NS
_ROWS_W = _ROWS // _NW
_CH_ROWS = 192
_NCHUNK = _ROWS_W // _CH_ROWS
_LANES = 16


def _tec_body(x_hbm, out_hbm, in0, in1, out0, out1,
              isem0, isem1, osem0, osem1):
    wid = lax.axis_index("s") * _NC + lax.axis_index("c")
    base = wid * _ROWS_W
    inbuf = (in0, in1)
    outbuf = (out0, out1)
    isem = (isem0, isem1)
    osem = (osem0, osem1)

    def load(c):
        b = c % 2
        return pltpu.async_copy(
            x_hbm.at[pl.ds(base + c * _CH_ROWS, _CH_ROWS)], inbuf[b], isem[b])

    def store(c):
        b = c % 2
        return pltpu.async_copy(
            outbuf[b], out_hbm.at[pl.ds(base + c * _CH_ROWS, _CH_ROWS)],
            osem[b])

    loads = {0: load(0)}
    if _NCHUNK > 1:
        loads[1] = load(1)
    stores = {}
    for c in range(_NCHUNK):
        b = c % 2
        loads[c].wait()
        if c >= 2:
            stores[c - 2].wait()
        src = inbuf[b]
        dst = outbuf[b]

        @plsc.parallel_loop(0, _CH_ROWS, unroll=4)
        def _row(r):
            for col in range(0, _COLS, _LANES):
                v = src[r, pl.ds(col, _LANES)]
                dst[r, pl.ds(col, _LANES)] = jnp.where(
                    v == 0.0, v, jnp.float32(2.0))

        stores[c] = store(c)
        if c + 2 < _NCHUNK:
            loads[c + 2] = load(c + 2)
    for c in range(max(0, _NCHUNK - 2), _NCHUNK):
        stores[c].wait()


@functools.partial(
    pl.kernel,
    mesh=plsc.VectorSubcoreMesh(core_axis_name="c", subcore_axis_name="s"),
    out_type=jax.ShapeDtypeStruct((_ROWS, _COLS), jnp.float32),
    scratch_types=[
        pltpu.VMEM((_CH_ROWS, _COLS), jnp.float32),
        pltpu.VMEM((_CH_ROWS, _COLS), jnp.float32),
        pltpu.VMEM((_CH_ROWS, _COLS), jnp.float32),
        pltpu.VMEM((_CH_ROWS, _COLS), jnp.float32),
        pltpu.SemaphoreType.DMA,
        pltpu.SemaphoreType.DMA,
        pltpu.SemaphoreType.DMA,
        pltpu.SemaphoreType.DMA,
    ],
)
def _sc_mask_set(x_hbm, out_hbm, *scratch):
    _tec_body(x_hbm, out_hbm, *scratch)


def kernel(x):
    rows = x.reshape(_ROWS, _COLS)
    out = _sc_mask_set(rows)
    return out.reshape(x.shape)
